# baseline (speedup 1.0000x reference)
import functools

import jax
import jax.numpy as jnp
from jax import lax
from jax.experimental import pallas as pl
from jax.experimental.pallas import tpu as pltpu
from jax.experimental.pallas import tpu_sc as plsc

NC = 2
NS = 16
NW = NC * NS
K = 80
NB = 4
NI = 8


def _spmm_sc(h, src, dst, np_rows):
    n_tab, d = h.shape
    e = src.shape[0]
    epw = e // NW
    n_chunks = epw // K
    assert epw * NW == e and n_chunks * K == epw
    rows_per_sub = np_rows // NS
    n_cp = rows_per_sub // K
    assert rows_per_sub * NS == np_rows and n_cp * K == rows_per_sub
    assert d % 16 == 0 and K % 8 == 0
    n_sg = (n_chunks - (NI - 1) - NB) // NI
    epi0 = n_sg * NI
    assert n_sg >= 1 and epi0 + NI <= n_chunks and n_chunks - epi0 >= NB

    mesh = plsc.VectorSubcoreMesh(
        core_axis_name="c", subcore_axis_name="s",
        num_cores=NC, num_subcores=NS)

    @functools.partial(
        pl.kernel,
        out_type=jax.ShapeDtypeStruct((NC, np_rows, d), jnp.float32),
        mesh=mesh,
        scratch_types=[
            pltpu.VMEM((NI, K), jnp.int32),
            pltpu.VMEM((NI, K), jnp.int32),
            pltpu.VMEM((NB, K, d), jnp.float32),
            pltpu.VMEM_SHARED((np_rows, d), jnp.float32),
            [pltpu.SemaphoreType.DMA] * NI,
            [pltpu.SemaphoreType.DMA] * NI,
            [pltpu.SemaphoreType.DMA] * NB,
            [pltpu.SemaphoreType.DMA] * NB,
        ],
    )
    def spmm(h_hbm, src_hbm, dst_hbm, out_hbm,
             sidx_v, didx_v, rows_v, acc_sh, sisem, disem, gsem, ssem):
        c = lax.axis_index("c")
        s = lax.axis_index("s")
        w = c * NS + s
        base = w * epw
        row0 = s * rows_per_sub

        def sidx_d(i, q):
            return pltpu.make_async_copy(
                src_hbm.at[pl.ds(base + i * K, K)], sidx_v.at[q], sisem[q])

        def didx_d(i, q):
            return pltpu.make_async_copy(
                dst_hbm.at[pl.ds(base + i * K, K)], didx_v.at[q], disem[q])

        def gather_d(b, q):
            return pltpu.make_async_copy(
                h_hbm.at[sidx_v.at[q]], rows_v.at[b], gsem[b])

        def scat_d(b, q):
            return pltpu.make_async_copy(
                rows_v.at[b], acc_sh.at[didx_v.at[q]], ssem[b])

        def scat_start(b, q):
            pltpu.async_copy(rows_v.at[b], acc_sh.at[didx_v.at[q]], ssem[b],
                             add=True)

        for q in range(NI):
            sidx_d(q, q).start()
            didx_d(q, q).start()

        zeros16 = jnp.zeros((16,), jnp.float32)

        def zrow(r, carry):
            def zcol(qq, inner):
                rows_v[r // K, r % K, pl.ds(qq * 16, 16)] = zeros16
                return inner
            return lax.fori_loop(0, d // 16, zcol, carry)

        lax.fori_loop(0, 2 * K, zrow, 0)
        for k in range(n_cp):
            pltpu.async_copy(rows_v.at[k % 2],
                             acc_sh.at[pl.ds(row0 + k * K, K)], gsem[k % 2])
        for k in range(n_cp):
            pltpu.make_async_copy(rows_v.at[k % 2],
                                  acc_sh.at[pl.ds(row0 + k * K, K)],
                                  gsem[k % 2]).wait()
        plsc.subcore_barrier()

        for b in range(NB):
            sidx_d(b, b).wait()
            gather_d(b, b).start()

        def supergroup(sg, carry):
            i0 = sg * NI
            for hh in range(NI // NB):
                for b in range(NB):
                    q = hh * NB + b
                    i = i0 + q
                    gather_d(b, q).wait()
                    didx_d(i, q).wait()
                    scat_start(b, q)
                for b in range(NB):
                    q = hh * NB + b
                    i = i0 + q
                    scat_d(b, q).wait()
                    sidx_d(i + NI, q).start()
                    didx_d(i + NI, q).start()
                    q2 = (q + NB) % NI
                    sidx_d(i + NB, q2).wait()
                    gather_d(b, q2).start()
            return carry

        lax.fori_loop(0, n_sg, supergroup, 0)

        for j in range(epi0, n_chunks):
            b, q = j % NB, j % NI
            gather_d(b, q).wait()
            didx_d(j, q).wait()
            scat_start(b, q)
            scat_d(b, q).wait()
            nj = j + NI
            if nj < n_chunks:
                sidx_d(nj, q).start()
                didx_d(nj, q).start()
            gj = j + NB
            if gj < n_chunks:
                q2 = gj % NI
                sidx_d(gj, q2).wait()
                gather_d(b, q2).start()
        plsc.subcore_barrier()

        def rd_d(k, b):
            return pltpu.make_async_copy(
                acc_sh.at[pl.ds(row0 + k * K, K)], rows_v.at[b], gsem[b])

        def wr_d(k, b):
            return pltpu.make_async_copy(
                rows_v.at[b], out_hbm.at[c, pl.ds(row0 + k * K, K)], ssem[b])

        for k in range(NB):
            rd_d(k, k).start()
        for k in range(n_cp):
            b = k % NB
            rd_d(k, b).wait()
            wr_d(k, b).start()
            kn = k + NB
            if kn < n_cp:
                wr_d(k, b).wait()
                rd_d(kn, b).start()
            else:
                wr_d(k, b).wait()

    return spmm(h, src, dst)


def _dense_tc(p, w, b, relu):
    nc, n, d = p.shape
    hdim = w.shape[1]
    br = 1280
    assert n % br == 0

    def body(p_ref, w_ref, b_ref, o_ref):
        agg = p_ref[0] + p_ref[1]
        z = jnp.dot(agg, w_ref[...], preferred_element_type=jnp.float32)
        z = z + b_ref[...]
        o_ref[...] = jnp.maximum(z, 0.0) if relu else z

    return pl.pallas_call(
        body,
        grid=(n // br,),
        in_specs=[
            pl.BlockSpec((nc, br, d), lambda i: (0, i, 0)),
            pl.BlockSpec((d, hdim), lambda i: (0, 0)),
            pl.BlockSpec((1, hdim), lambda i: (0, 0)),
        ],
        out_specs=pl.BlockSpec((br, hdim), lambda i: (i, 0)),
        out_shape=jax.ShapeDtypeStruct((n, hdim), jnp.float32),
    )(p, w, b.reshape(1, hdim))


def kernel(x, edge_index, W1, b1, W2, b2, W3, b3):
    n = x.shape[0]
    np_rows = ((n + 16 * 128 - 1) // (16 * 128)) * (16 * 128)
    ei = edge_index.astype(jnp.int32)
    src, dst = ei[0], ei[1]
    p = _spmm_sc(x, src, dst, np_rows)
    h1 = _dense_tc(p, W1, b1, True)
    p = _spmm_sc(h1, src, dst, np_rows)
    h2 = _dense_tc(p, W2, b2, True)
    p = _spmm_sc(h2, src, dst, np_rows)
    z = _dense_tc(p, W3, b3, False)
    return z[:n]


# speedup vs baseline: 1.0045x; 1.0045x over previous
import functools

import jax
import jax.numpy as jnp
from jax import lax
from jax.experimental import pallas as pl
from jax.experimental.pallas import tpu as pltpu
from jax.experimental.pallas import tpu_sc as plsc

NC = 2
NS = 16
NW = NC * NS
K = 80
NB = 4
NI = 8


def _spmm_sc(h, src, dst, np_rows):
    n_tab, d = h.shape
    e = src.shape[0]
    epw = e // NW
    n_chunks = epw // K
    assert epw * NW == e and n_chunks * K == epw
    rows_per_sub = np_rows // NS
    n_cp = rows_per_sub // K
    assert rows_per_sub * NS == np_rows and n_cp * K == rows_per_sub
    assert d % 16 == 0 and K % 8 == 0
    n_sg = (n_chunks - (NI - 1) - NB) // NI
    epi0 = n_sg * NI
    assert n_sg >= 1 and epi0 + NI <= n_chunks and n_chunks - epi0 >= NB

    mesh = plsc.VectorSubcoreMesh(
        core_axis_name="c", subcore_axis_name="s",
        num_cores=NC, num_subcores=NS)

    @functools.partial(
        pl.kernel,
        out_type=jax.ShapeDtypeStruct((NC, np_rows, d), jnp.float32),
        mesh=mesh,
        scratch_types=[
            pltpu.VMEM((NI, K), jnp.int32),
            pltpu.VMEM((NI, K), jnp.int32),
            pltpu.VMEM((NB, K, d), jnp.float32),
            pltpu.VMEM_SHARED((np_rows, d), jnp.float32),
            [pltpu.SemaphoreType.DMA] * NI,
            [pltpu.SemaphoreType.DMA] * NI,
            [pltpu.SemaphoreType.DMA] * NB,
            [pltpu.SemaphoreType.DMA] * NB,
        ],
    )
    def spmm(h_hbm, src_hbm, dst_hbm, out_hbm,
             sidx_v, didx_v, rows_v, acc_sh, sisem, disem, gsem, ssem):
        c = lax.axis_index("c")
        s = lax.axis_index("s")
        w = c * NS + s
        base = w * epw
        row0 = s * rows_per_sub

        def sidx_d(i, q):
            return pltpu.make_async_copy(
                src_hbm.at[pl.ds(base + i * K, K)], sidx_v.at[q], sisem[q])

        def didx_d(i, q):
            return pltpu.make_async_copy(
                dst_hbm.at[pl.ds(base + i * K, K)], didx_v.at[q], disem[q])

        def gather_d(b, q):
            return pltpu.make_async_copy(
                h_hbm.at[sidx_v.at[q]], rows_v.at[b], gsem[b])

        def scat_d(b, q):
            return pltpu.make_async_copy(
                rows_v.at[b], acc_sh.at[didx_v.at[q]], ssem[b])

        def scat_start(b, q):
            pltpu.async_copy(rows_v.at[b], acc_sh.at[didx_v.at[q]], ssem[b],
                             add=True)

        for q in range(NI):
            sidx_d(q, q).start()
            didx_d(q, q).start()

        zeros16 = jnp.zeros((16,), jnp.float32)

        def zrow(r, carry):
            def zcol(qq, inner):
                rows_v[0, r, pl.ds(qq * 16, 16)] = zeros16
                return inner
            return lax.fori_loop(0, d // 16, zcol, carry)

        lax.fori_loop(0, K, zrow, 0)
        for k in range(n_cp):
            pltpu.async_copy(rows_v.at[0],
                             acc_sh.at[pl.ds(row0 + k * K, K)], gsem[0])
        for k in range(n_cp):
            pltpu.make_async_copy(rows_v.at[0],
                                  acc_sh.at[pl.ds(row0 + k * K, K)],
                                  gsem[0]).wait()
        plsc.subcore_barrier()

        for b in range(NB):
            sidx_d(b, b).wait()
            gather_d(b, b).start()

        def supergroup(sg, carry):
            i0 = sg * NI
            for hh in range(NI // NB):
                for b in range(NB):
                    q = hh * NB + b
                    i = i0 + q
                    gather_d(b, q).wait()
                    didx_d(i, q).wait()
                    scat_start(b, q)
                for b in range(NB):
                    q = hh * NB + b
                    i = i0 + q
                    scat_d(b, q).wait()
                    sidx_d(i + NI, q).start()
                    didx_d(i + NI, q).start()
                    q2 = (q + NB) % NI
                    sidx_d(i + NB, q2).wait()
                    gather_d(b, q2).start()
            return carry

        lax.fori_loop(0, n_sg, supergroup, 0)

        for j in range(epi0, n_chunks):
            b, q = j % NB, j % NI
            gather_d(b, q).wait()
            didx_d(j, q).wait()
            scat_start(b, q)
            scat_d(b, q).wait()
            nj = j + NI
            if nj < n_chunks:
                sidx_d(nj, q).start()
                didx_d(nj, q).start()
            gj = j + NB
            if gj < n_chunks:
                q2 = gj % NI
                sidx_d(gj, q2).wait()
                gather_d(b, q2).start()
        plsc.subcore_barrier()

        def rd_d(k, b):
            return pltpu.make_async_copy(
                acc_sh.at[pl.ds(row0 + k * K, K)], rows_v.at[b], gsem[b])

        def wr_d(k, b):
            return pltpu.make_async_copy(
                rows_v.at[b], out_hbm.at[c, pl.ds(row0 + k * K, K)], ssem[b])

        for k in range(NB):
            rd_d(k, k).start()
        for k in range(n_cp):
            b = k % NB
            rd_d(k, b).wait()
            wr_d(k, b).start()
            kn = k + NB
            if kn < n_cp:
                wr_d(k, b).wait()
                rd_d(kn, b).start()
            else:
                wr_d(k, b).wait()

    return spmm(h, src, dst)


def _dense_tc(p, w, b, relu, out_n=None):
    nc, n, d = p.shape
    hdim = w.shape[1]
    out_n = n if out_n is None else out_n
    br = 1280 if out_n % 1280 == 0 else 1000
    assert out_n % br == 0

    def body(p_ref, w_ref, b_ref, o_ref):
        agg = p_ref[0] + p_ref[1]
        z = jnp.dot(agg, w_ref[...], preferred_element_type=jnp.float32)
        z = z + b_ref[...]
        o_ref[...] = jnp.maximum(z, 0.0) if relu else z

    return pl.pallas_call(
        body,
        grid=(out_n // br,),
        in_specs=[
            pl.BlockSpec((nc, br, d), lambda i: (0, i, 0)),
            pl.BlockSpec((d, hdim), lambda i: (0, 0)),
            pl.BlockSpec((1, hdim), lambda i: (0, 0)),
        ],
        out_specs=pl.BlockSpec((br, hdim), lambda i: (i, 0)),
        out_shape=jax.ShapeDtypeStruct((out_n, hdim), jnp.float32),
    )(p, w, b.reshape(1, hdim))


def kernel(x, edge_index, W1, b1, W2, b2, W3, b3):
    n = x.shape[0]
    np_rows = ((n + 16 * 128 - 1) // (16 * 128)) * (16 * 128)
    ei = edge_index.astype(jnp.int32)
    src, dst = ei[0], ei[1]
    p = _spmm_sc(x, src, dst, np_rows)
    h1 = _dense_tc(p, W1, b1, True)
    p = _spmm_sc(h1, src, dst, np_rows)
    h2 = _dense_tc(p, W2, b2, True)
    p = _spmm_sc(h2, src, dst, np_rows)
    return _dense_tc(p, W3, b3, False, out_n=n)


# speedup vs baseline: 1.0146x; 1.0101x over previous
import functools

import jax
import jax.numpy as jnp
from jax import lax
from jax.experimental import pallas as pl
from jax.experimental.pallas import tpu as pltpu
from jax.experimental.pallas import tpu_sc as plsc

NC = 2
NS = 16
NW = NC * NS
K = 80
NB = 4
NI = 8


def _spmm_sc(h, src, dst, np_rows):
    n_tab, d = h.shape
    e = src.shape[0]
    epw = e // NW
    n_chunks = epw // K
    assert epw * NW == e and n_chunks * K == epw
    rows_per_sub = np_rows // NS
    n_cp = rows_per_sub // K
    assert rows_per_sub * NS == np_rows and n_cp * K == rows_per_sub
    assert d % 16 == 0 and K % 8 == 0
    n_sg = (n_chunks - (NI - 1) - NB) // NI
    epi0 = n_sg * NI
    assert n_sg >= 1 and epi0 + NI <= n_chunks and n_chunks - epi0 >= NB

    mesh = plsc.VectorSubcoreMesh(
        core_axis_name="c", subcore_axis_name="s",
        num_cores=NC, num_subcores=NS)

    @functools.partial(
        pl.kernel,
        out_type=jax.ShapeDtypeStruct((NC, np_rows, d), jnp.float32),
        mesh=mesh,
        scratch_types=[
            pltpu.VMEM((NI, K), jnp.int32),
            pltpu.VMEM((NI, K), jnp.int32),
            pltpu.VMEM((NB, K, d), jnp.float32),
            pltpu.VMEM_SHARED((np_rows, d), jnp.float32),
            [pltpu.SemaphoreType.DMA] * NI,
            [pltpu.SemaphoreType.DMA] * NI,
            [pltpu.SemaphoreType.DMA] * NB,
            [pltpu.SemaphoreType.DMA] * NB,
            [pltpu.SemaphoreType.DMA] * NB,
        ],
    )
    def spmm(h_hbm, src_hbm, dst_hbm, out_hbm,
             sidx_v, didx_v, rows_v, acc_sh, sisem, disem, gsem, g2sem,
             ssem):
        c = lax.axis_index("c")
        s = lax.axis_index("s")
        w = c * NS + s
        base = w * epw
        row0 = s * rows_per_sub

        def sidx_d(i, q):
            return pltpu.make_async_copy(
                src_hbm.at[pl.ds(base + i * K, K)], sidx_v.at[q], sisem[q])

        def didx_d(i, q):
            return pltpu.make_async_copy(
                dst_hbm.at[pl.ds(base + i * K, K)], didx_v.at[q], disem[q])

        KH = K // 2

        def gather_lo(b, q):
            return pltpu.make_async_copy(
                h_hbm.at[sidx_v.at[q, pl.ds(0, KH)]],
                rows_v.at[b, pl.ds(0, KH)], gsem[b])

        def gather_hi(b, q):
            return pltpu.make_async_copy(
                h_hbm.at[sidx_v.at[q, pl.ds(KH, KH)]],
                rows_v.at[b, pl.ds(KH, KH)], g2sem[b])

        def gather_start(b, q):
            gather_lo(b, q).start()
            gather_hi(b, q).start()

        def gather_wait(b, q):
            gather_lo(b, q).wait()
            gather_hi(b, q).wait()

        def scat_d(b, q):
            return pltpu.make_async_copy(
                rows_v.at[b], acc_sh.at[didx_v.at[q]], ssem[b])

        def scat_start(b, q):
            pltpu.async_copy(rows_v.at[b], acc_sh.at[didx_v.at[q]], ssem[b],
                             add=True)

        for q in range(NI):
            sidx_d(q, q).start()
            didx_d(q, q).start()

        zeros16 = jnp.zeros((16,), jnp.float32)

        def zrow(r, carry):
            def zcol(qq, inner):
                rows_v[0, r, pl.ds(qq * 16, 16)] = zeros16
                return inner
            return lax.fori_loop(0, d // 16, zcol, carry)

        lax.fori_loop(0, K, zrow, 0)
        for k in range(n_cp):
            pltpu.async_copy(rows_v.at[0],
                             acc_sh.at[pl.ds(row0 + k * K, K)], gsem[0])
        for k in range(n_cp):
            pltpu.make_async_copy(rows_v.at[0],
                                  acc_sh.at[pl.ds(row0 + k * K, K)],
                                  gsem[0]).wait()
        plsc.subcore_barrier()

        for b in range(NB):
            sidx_d(b, b).wait()
            gather_start(b, b)

        def supergroup(sg, carry):
            i0 = sg * NI
            for hh in range(NI // NB):
                for b in range(NB):
                    q = hh * NB + b
                    i = i0 + q
                    gather_wait(b, q)
                    didx_d(i, q).wait()
                    scat_start(b, q)
                for b in range(NB):
                    q = hh * NB + b
                    i = i0 + q
                    scat_d(b, q).wait()
                    sidx_d(i + NI, q).start()
                    didx_d(i + NI, q).start()
                    q2 = (q + NB) % NI
                    sidx_d(i + NB, q2).wait()
                    gather_start(b, q2)
            return carry

        lax.fori_loop(0, n_sg, supergroup, 0)

        for j in range(epi0, n_chunks):
            b, q = j % NB, j % NI
            gather_wait(b, q)
            didx_d(j, q).wait()
            scat_start(b, q)
            scat_d(b, q).wait()
            nj = j + NI
            if nj < n_chunks:
                sidx_d(nj, q).start()
                didx_d(nj, q).start()
            gj = j + NB
            if gj < n_chunks:
                q2 = gj % NI
                sidx_d(gj, q2).wait()
                gather_start(b, q2)
        plsc.subcore_barrier()

        def rd_d(k, b):
            return pltpu.make_async_copy(
                acc_sh.at[pl.ds(row0 + k * K, K)], rows_v.at[b], gsem[b])

        def wr_d(k, b):
            return pltpu.make_async_copy(
                rows_v.at[b], out_hbm.at[c, pl.ds(row0 + k * K, K)], ssem[b])

        for k in range(NB):
            rd_d(k, k).start()
        for k in range(n_cp):
            b = k % NB
            rd_d(k, b).wait()
            wr_d(k, b).start()
            kn = k + NB
            if kn < n_cp:
                wr_d(k, b).wait()
                rd_d(kn, b).start()
            else:
                wr_d(k, b).wait()

    return spmm(h, src, dst)


def _dense_tc(p, w, b, relu, out_n=None):
    nc, n, d = p.shape
    hdim = w.shape[1]
    out_n = n if out_n is None else out_n
    br = 1280 if out_n % 1280 == 0 else 1000
    assert out_n % br == 0

    def body(p_ref, w_ref, b_ref, o_ref):
        agg = p_ref[0] + p_ref[1]
        z = jnp.dot(agg, w_ref[...], preferred_element_type=jnp.float32)
        z = z + b_ref[...]
        o_ref[...] = jnp.maximum(z, 0.0) if relu else z

    return pl.pallas_call(
        body,
        grid=(out_n // br,),
        in_specs=[
            pl.BlockSpec((nc, br, d), lambda i: (0, i, 0)),
            pl.BlockSpec((d, hdim), lambda i: (0, 0)),
            pl.BlockSpec((1, hdim), lambda i: (0, 0)),
        ],
        out_specs=pl.BlockSpec((br, hdim), lambda i: (i, 0)),
        out_shape=jax.ShapeDtypeStruct((out_n, hdim), jnp.float32),
    )(p, w, b.reshape(1, hdim))


def kernel(x, edge_index, W1, b1, W2, b2, W3, b3):
    n = x.shape[0]
    np_rows = ((n + 16 * 128 - 1) // (16 * 128)) * (16 * 128)
    ei = edge_index.astype(jnp.int32)
    src, dst = ei[0], ei[1]
    p = _spmm_sc(x, src, dst, np_rows)
    h1 = _dense_tc(p, W1, b1, True)
    p = _spmm_sc(h1, src, dst, np_rows)
    h2 = _dense_tc(p, W2, b2, True)
    p = _spmm_sc(h2, src, dst, np_rows)
    return _dense_tc(p, W3, b3, False, out_n=n)


# speedup vs baseline: 1.0149x; 1.0003x over previous
import functools

import jax
import jax.numpy as jnp
from jax import lax
from jax.experimental import pallas as pl
from jax.experimental.pallas import tpu as pltpu
from jax.experimental.pallas import tpu_sc as plsc

NC = 2
NS = 16
NW = NC * NS
K = 80
NB = 4
NI = 8


def _spmm_sc(h, src, dst, np_rows):
    n_tab, d = h.shape
    e = src.shape[0]
    epw = e // NW
    n_chunks = epw // K
    assert epw * NW == e and n_chunks * K == epw
    rows_per_sub = np_rows // NS
    n_cp = rows_per_sub // K
    assert rows_per_sub * NS == np_rows and n_cp * K == rows_per_sub
    assert d % 16 == 0 and K % 8 == 0
    n_sg = (n_chunks - (NI - 1) - NB) // NI
    epi0 = n_sg * NI
    assert n_sg >= 1 and epi0 + NI <= n_chunks and n_chunks - epi0 >= NB

    mesh = plsc.VectorSubcoreMesh(
        core_axis_name="c", subcore_axis_name="s",
        num_cores=NC, num_subcores=NS)

    @functools.partial(
        pl.kernel,
        out_type=jax.ShapeDtypeStruct((NC, np_rows, d), jnp.float32),
        mesh=mesh,
        scratch_types=[
            pltpu.VMEM((NI, K), jnp.int32),
            pltpu.VMEM((NI, K), jnp.int32),
            pltpu.VMEM((NB, K, d), jnp.float32),
            pltpu.VMEM_SHARED((np_rows, d), jnp.float32),
            [pltpu.SemaphoreType.DMA] * NI,
            [pltpu.SemaphoreType.DMA] * NI,
            [pltpu.SemaphoreType.DMA] * NB,
            [pltpu.SemaphoreType.DMA] * NB,
            [pltpu.SemaphoreType.DMA] * NB,
        ],
    )
    def spmm(h_hbm, src_hbm, dst_hbm, out_hbm,
             sidx_v, didx_v, rows_v, acc_sh, sisem, disem, gsem, g2sem,
             ssem):
        c = lax.axis_index("c")
        s = lax.axis_index("s")
        w = c * NS + s
        base = w * epw
        row0 = s * rows_per_sub

        def sidx_d(i, q):
            return pltpu.make_async_copy(
                src_hbm.at[pl.ds(base + i * K, K)], sidx_v.at[q], sisem[q])

        def didx_d(i, q):
            return pltpu.make_async_copy(
                dst_hbm.at[pl.ds(base + i * K, K)], didx_v.at[q], disem[q])

        KH = K // 2

        def gather_lo(b, q):
            return pltpu.make_async_copy(
                h_hbm.at[sidx_v.at[q, pl.ds(0, KH)]],
                rows_v.at[b, pl.ds(0, KH)], gsem[b])

        def gather_hi(b, q):
            return pltpu.make_async_copy(
                h_hbm.at[sidx_v.at[q, pl.ds(KH, KH)]],
                rows_v.at[b, pl.ds(KH, KH)], g2sem[b])

        def gather_start(b, q):
            gather_lo(b, q).start()
            gather_hi(b, q).start()

        def gather_wait(b, q):
            gather_lo(b, q).wait()
            gather_hi(b, q).wait()

        def scat_d(b, q):
            return pltpu.make_async_copy(
                rows_v.at[b], acc_sh.at[didx_v.at[q]], ssem[b])

        def scat_start(b, q):
            pltpu.async_copy(rows_v.at[b], acc_sh.at[didx_v.at[q]], ssem[b],
                             add=True)

        for q in range(NI):
            sidx_d(q, q).start()
            didx_d(q, q).start()

        zeros16 = jnp.zeros((16,), jnp.float32)

        def zrow(r, carry):
            def zcol(qq, inner):
                rows_v[0, r, pl.ds(qq * 16, 16)] = zeros16
                return inner
            return lax.fori_loop(0, d // 16, zcol, carry)

        lax.fori_loop(0, K, zrow, 0)
        for k in range(n_cp):
            pltpu.async_copy(rows_v.at[0],
                             acc_sh.at[pl.ds(row0 + k * K, K)], gsem[k % NB])
        for k in range(n_cp):
            pltpu.make_async_copy(rows_v.at[0],
                                  acc_sh.at[pl.ds(row0 + k * K, K)],
                                  gsem[k % NB]).wait()
        plsc.subcore_barrier()

        for b in range(NB):
            sidx_d(b, b).wait()
            gather_start(b, b)

        def supergroup(sg, carry):
            i0 = sg * NI
            for hh in range(NI // NB):
                for b in range(NB):
                    q = hh * NB + b
                    i = i0 + q
                    gather_wait(b, q)
                    didx_d(i, q).wait()
                    scat_start(b, q)
                for b in range(NB):
                    q = hh * NB + b
                    i = i0 + q
                    scat_d(b, q).wait()
                    sidx_d(i + NI, q).start()
                    didx_d(i + NI, q).start()
                    q2 = (q + NB) % NI
                    sidx_d(i + NB, q2).wait()
                    gather_start(b, q2)
            return carry

        lax.fori_loop(0, n_sg, supergroup, 0)

        for j in range(epi0, n_chunks):
            b, q = j % NB, j % NI
            gather_wait(b, q)
            didx_d(j, q).wait()
            scat_start(b, q)
            nj = j + NI
            if nj < n_chunks:
                sidx_d(nj, q).start()
                didx_d(nj, q).start()
            gj = j + NB
            if gj < n_chunks:
                q2 = gj % NI
                scat_d(b, q).wait()
                sidx_d(gj, q2).wait()
                gather_start(b, q2)
        for j in range(max(epi0, n_chunks - NB), n_chunks):
            b, q = j % NB, j % NI
            scat_d(b, q).wait()
        plsc.subcore_barrier()

        def rd_d(k, b):
            return pltpu.make_async_copy(
                acc_sh.at[pl.ds(row0 + k * K, K)], rows_v.at[b], gsem[b])

        def wr_d(k, b):
            return pltpu.make_async_copy(
                rows_v.at[b], out_hbm.at[c, pl.ds(row0 + k * K, K)], ssem[b])

        for k in range(NB):
            rd_d(k, k).start()
        for k in range(n_cp):
            b = k % NB
            rd_d(k, b).wait()
            wr_d(k, b).start()
            kn = k + NB
            if kn < n_cp:
                wr_d(k, b).wait()
                rd_d(kn, b).start()
            else:
                wr_d(k, b).wait()

    return spmm(h, src, dst)


def _dense_tc(p, w, b, relu, out_n=None):
    nc, n, d = p.shape
    hdim = w.shape[1]
    out_n = n if out_n is None else out_n
    br = 1280 if out_n % 1280 == 0 else 1000
    assert out_n % br == 0

    def body(p_ref, w_ref, b_ref, o_ref):
        agg = p_ref[0] + p_ref[1]
        z = jnp.dot(agg, w_ref[...], preferred_element_type=jnp.float32)
        z = z + b_ref[...]
        o_ref[...] = jnp.maximum(z, 0.0) if relu else z

    return pl.pallas_call(
        body,
        grid=(out_n // br,),
        in_specs=[
            pl.BlockSpec((nc, br, d), lambda i: (0, i, 0)),
            pl.BlockSpec((d, hdim), lambda i: (0, 0)),
            pl.BlockSpec((1, hdim), lambda i: (0, 0)),
        ],
        out_specs=pl.BlockSpec((br, hdim), lambda i: (i, 0)),
        out_shape=jax.ShapeDtypeStruct((out_n, hdim), jnp.float32),
    )(p, w, b.reshape(1, hdim))


def kernel(x, edge_index, W1, b1, W2, b2, W3, b3):
    n = x.shape[0]
    np_rows = ((n + 16 * 128 - 1) // (16 * 128)) * (16 * 128)
    ei = edge_index.astype(jnp.int32)
    src, dst = ei[0], ei[1]
    p = _spmm_sc(x, src, dst, np_rows)
    h1 = _dense_tc(p, W1, b1, True)
    p = _spmm_sc(h1, src, dst, np_rows)
    h2 = _dense_tc(p, W2, b2, True)
    p = _spmm_sc(h2, src, dst, np_rows)
    return _dense_tc(p, W3, b3, False, out_n=n)
